# baseline (speedup 1.0000x reference)
import functools

import jax
import jax.numpy as jnp
from jax import lax
from jax.experimental import pallas as pl
from jax.experimental.pallas import tpu as pltpu
from jax.experimental.pallas import tpu_sc as plsc

L = 16
NC = 2
NS = 16
NW = NC * NS
B = 16384
D = 32
V = 1000000
BPW = B // NW

WIN = 1024
NFULL = V // WIN
WPW = NFULL // NW
NEXTRA = NFULL - WPW * NW
XBASE = WPW * WIN * NW
HBLK = XBASE + NEXTRA * WIN
TAIL0 = 999936
TAILW = V - TAIL0
STAGE_ROWS = B + 2 * NW

CHUNK = 2048
SEG = 2048

_mesh = plsc.VectorSubcoreMesh(core_axis_name="c", subcore_axis_name="s")

_GATHER_DNUMS = lax.GatherDimensionNumbers(
    offset_dims=(), collapsed_slice_dims=(0,), start_index_map=(0,))


def _lane_shuffle(x, perm):
    return lax.gather(x, perm[:, None], _GATHER_DNUMS, slice_sizes=(1,),
                      mode=lax.GatherScatterMode.PROMISE_IN_BOUNDS)


def _iota():
    return lax.iota(jnp.int32, L)


@functools.partial(
    pl.kernel,
    mesh=_mesh,
    out_type=(jax.ShapeDtypeStruct((STAGE_ROWS, 128), jnp.float32),
              jax.ShapeDtypeStruct((STAGE_ROWS, 128), jnp.float32)),
    compiler_params=pltpu.CompilerParams(use_tc_tiling_on_sc=True,
                                         needs_layout_passes=False),
    scratch_types=[
        pltpu.VMEM((CHUNK,), jnp.int32),
        pltpu.VMEM((B,), jnp.int32),
        pltpu.VMEM((B,), jnp.int32),
        pltpu.VMEM((2, D, WIN + 1), jnp.float32),
        pltpu.VMEM((D, 128), jnp.float32),
        pltpu.VMEM((D, 128), jnp.float32),
        pltpu.VMEM((SEG + L,), jnp.int32),
        pltpu.VMEM((SEG + L,), jnp.int32),
        pltpu.VMEM((64, 128), jnp.float32),
        pltpu.VMEM((64,), jnp.int32),
        pltpu.SemaphoreType.DMA,
        pltpu.SemaphoreType.DMA,
    ],
)
def _sweep_kernel(uid_hbm, mid_hbm, uemb_hbm, memb_hbm, tailu_hbm, tailm_hbm,
                  stage_u, stage_m, chunk_v, list_id, list_pos, win_v,
                  tailu_v, tailm_v, wl_id, wl_pos, rows_v, pos_v, wsem, ssem):
    wid = lax.axis_index("s") * NC + lax.axis_index("c")
    iot = _iota()
    lo = WPW * WIN * wid
    hi = lo + WPW * WIN
    xlo = jnp.where(wid < NEXTRA, XBASE + wid * WIN,
                    jnp.where(wid == NEXTRA, HBLK,
                              jnp.where(wid == NW - 1, TAIL0, 0)))
    xhi = jnp.where(wid < NEXTRA, XBASE + wid * WIN + WIN,
                    jnp.where(wid == NEXTRA, TAIL0,
                              jnp.where(wid == NW - 1, V, 0)))
    trash = B + wid

    pltpu.sync_copy(tailu_hbm, tailu_v)
    pltpu.sync_copy(tailm_hbm, tailm_v)

    def build_list(ids_hbm):
        def chunk_body(ci, off):
            pltpu.sync_copy(ids_hbm.at[pl.ds(ci * CHUNK, CHUNK)], chunk_v)

            def vreg_body(i, off):
                v = chunk_v[pl.ds(i * L, L)]
                posv = ci * CHUNK + i * L + iot
                m = ((v >= lo) & (v < hi)) | ((v >= xlo) & (v < xhi))
                plsc.store_compressed(list_id.at[pl.ds(off, L)], v, mask=m)
                plsc.store_compressed(list_pos.at[pl.ds(off, L)], posv, mask=m)
                return off + plsc.all_reduce_population_count(m)[0]

            return lax.fori_loop(0, CHUNK // L, vreg_body, off)

        return lax.fori_loop(0, B // CHUNK, chunk_body, 0)

    def flush(stage):
        pltpu.async_copy(rows_v, stage.at[pos_v], ssem).wait()

    def process_window(src_v, c0, width, cnt, stage, fc):
        nseg = (cnt + (SEG - 1)) // SEG

        def seg_body(s, fc):
            sbase = s * SEG

            def sv(i, woff):
                gi = sbase + i * L
                v = list_id[pl.ds(gi, L)]
                p = list_pos[pl.ds(gi, L)]
                m = ((gi + iot) < cnt) & (v >= c0) & (v < c0 + width)
                plsc.store_compressed(wl_id.at[pl.ds(woff, L)], v - c0, mask=m)
                plsc.store_compressed(wl_pos.at[pl.ds(woff, L)], p, mask=m)
                return woff + plsc.all_reduce_population_count(m)[0]

            nv = (jnp.minimum(cnt - sbase, SEG) + (L - 1)) // L
            wcnt = lax.fori_loop(0, nv, sv, 0)
            plsc.store_scatter(wl_id, [wcnt + iot], jnp.zeros((L,), jnp.int32),
                               mask=None)
            plsc.store_scatter(wl_pos, [wcnt + iot],
                               jnp.full((L,), trash, jnp.int32), mask=None)

            def grp_body(g, fc):
                wc16 = wl_id[pl.ds(g * L, L)]
                pos16 = wl_pos[pl.ds(g * L, L)]
                pos_v[pl.ds(fc * L, L)] = pos16
                for j in range(L):
                    wcj = _lane_shuffle(wc16, jnp.full((L,), j, jnp.int32))
                    g0 = plsc.load_gather(src_v, [iot, wcj])
                    g1 = plsc.load_gather(src_v, [iot + L, wcj])
                    r = fc * L + j
                    rows_v[r, pl.ds(0, L)] = g0
                    rows_v[r, pl.ds(L, L)] = g1

                @pl.when(fc == 3)
                def _():
                    flush(stage)

                return (fc + 1) & 3

            ngrp = (wcnt + (L - 1)) // L
            return lax.fori_loop(0, ngrp, grp_body, fc)

        return lax.fori_loop(0, nseg, seg_body, fc)

    def sweep_table(ids_hbm, emb_hbm, stage, tail_ref):
        for q in range(4):
            pos_v[pl.ds(q * L, L)] = jnp.full((L,), trash, jnp.int32)

        for b in range(2):
            cb = pl.multiple_of(lo + b * WIN, WIN)
            pltpu.async_copy(emb_hbm.at[:, pl.ds(cb, WIN)],
                             win_v.at[b, :, pl.ds(0, WIN)], wsem)

        cnt = build_list(ids_hbm)

        def pair_body(p, fc):
            for b in range(2):
                k = 2 * p + b
                cw = pl.multiple_of(lo + k * WIN, WIN)
                pltpu.make_async_copy(emb_hbm.at[:, pl.ds(cw, WIN)],
                                      win_v.at[b, :, pl.ds(0, WIN)],
                                      wsem).wait()
                fc = process_window(win_v.at[b], cw, WIN, cnt, stage, fc)

                k2 = k + 2

                @pl.when(k2 < WPW)
                def _():
                    c2 = pl.multiple_of(lo + k2 * WIN, WIN)
                    pltpu.async_copy(emb_hbm.at[:, pl.ds(c2, WIN)],
                                     win_v.at[b, :, pl.ds(0, WIN)], wsem)

            return fc

        fc = lax.fori_loop(0, WPW // 2, pair_body, jnp.int32(0))

        cx = pl.multiple_of(
            jnp.where(wid < NEXTRA, XBASE + wid * WIN, XBASE), WIN)
        pltpu.sync_copy(emb_hbm.at[:, pl.ds(cx, WIN)],
                        win_v.at[0, :, pl.ds(0, WIN)])
        fc = process_window(win_v.at[0], cx, WIN, cnt, stage, fc)
        cy = pl.multiple_of(HBLK, 512)
        pltpu.sync_copy(emb_hbm.at[:, pl.ds(cy, 512)],
                        win_v.at[0, :, pl.ds(0, 512)])
        fc = process_window(win_v.at[0], cy, 512, cnt, stage, fc)
        fc = process_window(tail_ref, TAIL0, TAILW, cnt, stage, fc)
        flush(stage)

    sweep_table(uid_hbm, uemb_hbm, stage_u, tailu_v)
    sweep_table(mid_hbm, memb_hbm, stage_m, tailm_v)


@functools.partial(
    pl.kernel,
    mesh=_mesh,
    out_type=jax.ShapeDtypeStruct((B,), jnp.float32),
    compiler_params=pltpu.CompilerParams(use_tc_tiling_on_sc=False),
    scratch_types=[
        pltpu.VMEM((BPW,), jnp.int32),
        pltpu.VMEM((BPW,), jnp.int32),
        pltpu.VMEM((BPW,), jnp.float32),
        pltpu.VMEM((BPW,), jnp.float32),
        pltpu.SemaphoreType.DMA,
    ],
)
def _bias_kernel(uid_hbm, mid_hbm, ubias_hbm, mbias_hbm, out_hbm,
                 uid_v, mid_v, ub_v, mb_v, sem):
    wid = lax.axis_index("s") * NC + lax.axis_index("c")
    base = wid * BPW
    pltpu.sync_copy(uid_hbm.at[pl.ds(base, BPW)], uid_v)
    pltpu.sync_copy(mid_hbm.at[pl.ds(base, BPW)], mid_v)
    c1 = pltpu.async_copy(ubias_hbm.at[uid_v], ub_v, sem)
    c2 = pltpu.async_copy(mbias_hbm.at[mid_v], mb_v, sem)
    c1.wait()
    c2.wait()

    def body(g, _):
        sl = pl.ds(g * L, L)
        ub_v[sl] = ub_v[sl] + mb_v[sl]
        return 0

    lax.fori_loop(0, BPW // L, body, 0)
    pltpu.sync_copy(ub_v, out_hbm.at[pl.ds(base, BPW)])


_RCH = 128


@functools.partial(
    pl.kernel,
    mesh=_mesh,
    out_type=jax.ShapeDtypeStruct((B,), jnp.float32),
    compiler_params=pltpu.CompilerParams(use_tc_tiling_on_sc=True),
    scratch_types=[
        pltpu.VMEM((_RCH, 128), jnp.float32),
        pltpu.VMEM((_RCH, 128), jnp.float32),
        pltpu.VMEM((BPW,), jnp.float32),
        pltpu.VMEM((BPW,), jnp.float32),
    ],
)
def _dot_kernel(stage_u, stage_m, bias_hbm, out_hbm, su_v, sm_v, bias_v,
                out_v):
    wid = lax.axis_index("s") * NC + lax.axis_index("c")
    base = wid * BPW
    iot = _iota()
    pltpu.sync_copy(bias_hbm.at[pl.ds(base, BPW)], bias_v)
    lane_masks = [iot == j for j in range(L)]

    def chunk_body(ci, _):
        pltpu.sync_copy(stage_u.at[pl.ds(base + ci * _RCH, _RCH)], su_v)
        pltpu.sync_copy(stage_m.at[pl.ds(base + ci * _RCH, _RCH)], sm_v)

        def grp_body(g, _):
            acc = jnp.zeros((L,), jnp.float32)
            for j in range(L):
                r = g * L + j
                p = (su_v[r, pl.ds(0, L)] * sm_v[r, pl.ds(0, L)]
                     + su_v[r, pl.ds(L, L)] * sm_v[r, pl.ds(L, L)])
                for sh in (8, 4, 2, 1):
                    p = p + _lane_shuffle(p, iot ^ sh)
                acc = jnp.where(lane_masks[j], p, acc)
            sl = pl.ds(ci * _RCH + g * L, L)
            out_v[sl] = acc + bias_v[sl]
            return 0

        lax.fori_loop(0, _RCH // L, grp_body, 0)
        return 0

    lax.fori_loop(0, BPW // _RCH, chunk_body, 0)
    pltpu.sync_copy(out_v, out_hbm.at[pl.ds(base, BPW)])


def kernel(user_ids, movie_ids, user_embedding, movie_embedding,
           user_bias, movie_bias):
    ut = user_embedding.T
    mt = movie_embedding.T
    pad = ((0, 0), (0, 128 - TAILW))
    tail_u = jnp.pad(user_embedding[TAIL0:, :].T, pad)
    tail_m = jnp.pad(movie_embedding[TAIL0:, :].T, pad)
    stage_u, stage_m = _sweep_kernel(user_ids, movie_ids, ut, mt,
                                     tail_u, tail_m)
    bias_sum = _bias_kernel(user_ids, movie_ids, user_bias.reshape(-1),
                            movie_bias.reshape(-1))
    return _dot_kernel(stage_u, stage_m, bias_sum)


# speedup vs baseline: 1.0298x; 1.0298x over previous
import functools

import jax
import jax.numpy as jnp
from jax import lax
from jax.experimental import pallas as pl
from jax.experimental.pallas import tpu as pltpu
from jax.experimental.pallas import tpu_sc as plsc

L = 16
NC = 2
NS = 16
NW = NC * NS
B = 16384
D = 32
V = 1000000
BPW = B // NW

WIN = 1024
NFULL = V // WIN
WPW = NFULL // NW
NEXTRA = NFULL - WPW * NW
XBASE = WPW * WIN * NW
HBLK = XBASE + NEXTRA * WIN
TAIL0 = 999936
TAILW = V - TAIL0
STAGE_ROWS = B + 2 * NW

CHUNK = 2048
SEG = 1024

_mesh = plsc.VectorSubcoreMesh(core_axis_name="c", subcore_axis_name="s")

_GATHER_DNUMS = lax.GatherDimensionNumbers(
    offset_dims=(), collapsed_slice_dims=(0,), start_index_map=(0,))


def _lane_shuffle(x, perm):
    return lax.gather(x, perm[:, None], _GATHER_DNUMS, slice_sizes=(1,),
                      mode=lax.GatherScatterMode.PROMISE_IN_BOUNDS)


def _iota():
    return lax.iota(jnp.int32, L)


@functools.partial(
    pl.kernel,
    mesh=_mesh,
    out_type=(jax.ShapeDtypeStruct((STAGE_ROWS, 128), jnp.float32),
              jax.ShapeDtypeStruct((STAGE_ROWS, 128), jnp.float32)),
    compiler_params=pltpu.CompilerParams(use_tc_tiling_on_sc=True,
                                         needs_layout_passes=False),
    scratch_types=[
        pltpu.VMEM((2, CHUNK), jnp.int32),
        pltpu.VMEM((B,), jnp.int32),
        pltpu.VMEM((B,), jnp.int32),
        pltpu.VMEM((2, D, WIN + 1), jnp.float32),
        pltpu.VMEM((D, 128), jnp.float32),
        pltpu.VMEM((D, 128), jnp.float32),
        pltpu.VMEM((SEG + L,), jnp.int32),
        pltpu.VMEM((SEG + L,), jnp.int32),
        pltpu.VMEM((64, 128), jnp.float32),
        pltpu.VMEM((64,), jnp.int32),
        pltpu.SemaphoreType.DMA,
        pltpu.SemaphoreType.DMA,
        pltpu.SemaphoreType.DMA,
    ],
)
def _sweep_kernel(uid_hbm, mid_hbm, uemb_hbm, memb_hbm, tailu_hbm, tailm_hbm,
                  stage_u, stage_m, chunk_v, list_id, list_pos, win_v,
                  tailu_v, tailm_v, wl_id, wl_pos, rows_v, pos_v, wsem, ssem,
                  csem):
    wid = lax.axis_index("s") * NC + lax.axis_index("c")
    iot = _iota()
    lo = WPW * WIN * wid
    hi = lo + WPW * WIN
    xlo = jnp.where(wid < NEXTRA, XBASE + wid * WIN,
                    jnp.where(wid == NEXTRA, HBLK,
                              jnp.where(wid == NW - 1, TAIL0, 0)))
    xhi = jnp.where(wid < NEXTRA, XBASE + wid * WIN + WIN,
                    jnp.where(wid == NEXTRA, TAIL0,
                              jnp.where(wid == NW - 1, V, 0)))
    trash = B + wid

    pltpu.sync_copy(tailu_hbm, tailu_v)
    pltpu.sync_copy(tailm_hbm, tailm_v)

    def build_list(ids_hbm):
        for b in range(2):
            pltpu.async_copy(ids_hbm.at[pl.ds(b * CHUNK, CHUNK)],
                             chunk_v.at[b], csem)

        def chunk_pair(cp, off):
            for b in range(2):
                ci = cp * 2 + b
                pltpu.make_async_copy(ids_hbm.at[pl.ds(ci * CHUNK, CHUNK)],
                                      chunk_v.at[b], csem).wait()

                def vreg_body(i, off):
                    v = chunk_v[b, pl.ds(i * L, L)]
                    posv = ci * CHUNK + i * L + iot
                    m = ((v >= lo) & (v < hi)) | ((v >= xlo) & (v < xhi))
                    plsc.store_compressed(list_id.at[pl.ds(off, L)], v, mask=m)
                    plsc.store_compressed(list_pos.at[pl.ds(off, L)], posv,
                                          mask=m)
                    return off + plsc.all_reduce_population_count(m)[0]

                off = lax.fori_loop(0, CHUNK // L, vreg_body, off)
                ci2 = ci + 2

                @pl.when(ci2 < B // CHUNK)
                def _():
                    pltpu.async_copy(ids_hbm.at[pl.ds(ci2 * CHUNK, CHUNK)],
                                     chunk_v.at[b], csem)

            return off

        return lax.fori_loop(0, B // CHUNK // 2, chunk_pair, 0)

    def flush(stage):
        pltpu.async_copy(rows_v, stage.at[pos_v], ssem).wait()

    def process_window(src_v, c0, width, cnt, stage, fc):
        nseg = (cnt + (SEG - 1)) // SEG

        def seg_body(s, fc):
            sbase = s * SEG

            def sv(i, woff):
                gi = sbase + i * L
                v = list_id[pl.ds(gi, L)]
                p = list_pos[pl.ds(gi, L)]
                m = ((gi + iot) < cnt) & (v >= c0) & (v < c0 + width)
                plsc.store_compressed(wl_id.at[pl.ds(woff, L)], v - c0, mask=m)
                plsc.store_compressed(wl_pos.at[pl.ds(woff, L)], p, mask=m)
                return woff + plsc.all_reduce_population_count(m)[0]

            nv = (jnp.minimum(cnt - sbase, SEG) + (L - 1)) // L
            wcnt = lax.fori_loop(0, nv, sv, 0)
            plsc.store_scatter(wl_id, [wcnt + iot], jnp.zeros((L,), jnp.int32),
                               mask=None)
            plsc.store_scatter(wl_pos, [wcnt + iot],
                               jnp.full((L,), trash, jnp.int32), mask=None)

            def grp_body(g, fc):
                wc16 = wl_id[pl.ds(g * L, L)]
                pos16 = wl_pos[pl.ds(g * L, L)]
                pos_v[pl.ds(fc * L, L)] = pos16
                for j in range(L):
                    wcj = _lane_shuffle(wc16, jnp.full((L,), j, jnp.int32))
                    g0 = plsc.load_gather(src_v, [iot, wcj])
                    g1 = plsc.load_gather(src_v, [iot + L, wcj])
                    r = fc * L + j
                    rows_v[r, pl.ds(0, L)] = g0
                    rows_v[r, pl.ds(L, L)] = g1

                @pl.when(fc == 3)
                def _():
                    flush(stage)

                return (fc + 1) & 3

            ngrp = (wcnt + (L - 1)) // L
            return lax.fori_loop(0, ngrp, grp_body, fc)

        return lax.fori_loop(0, nseg, seg_body, fc)

    def sweep_table(ids_hbm, emb_hbm, stage, tail_ref):
        for q in range(4):
            pos_v[pl.ds(q * L, L)] = jnp.full((L,), trash, jnp.int32)

        for b in range(2):
            cb = pl.multiple_of(lo + b * WIN, WIN)
            pltpu.async_copy(emb_hbm.at[:, pl.ds(cb, WIN)],
                             win_v.at[b, :, pl.ds(0, WIN)], wsem)

        cnt = build_list(ids_hbm)

        def pair_body(p, fc):
            for b in range(2):
                k = 2 * p + b
                cw = pl.multiple_of(lo + k * WIN, WIN)
                pltpu.make_async_copy(emb_hbm.at[:, pl.ds(cw, WIN)],
                                      win_v.at[b, :, pl.ds(0, WIN)],
                                      wsem).wait()
                fc = process_window(win_v.at[b], cw, WIN, cnt, stage, fc)

                k2 = k + 2

                @pl.when(k2 < WPW)
                def _():
                    c2 = pl.multiple_of(lo + k2 * WIN, WIN)
                    pltpu.async_copy(emb_hbm.at[:, pl.ds(c2, WIN)],
                                     win_v.at[b, :, pl.ds(0, WIN)], wsem)

            return fc

        fc = lax.fori_loop(0, WPW // 2, pair_body, jnp.int32(0))

        cx = pl.multiple_of(
            jnp.where(wid < NEXTRA, XBASE + wid * WIN, XBASE), WIN)
        pltpu.sync_copy(emb_hbm.at[:, pl.ds(cx, WIN)],
                        win_v.at[0, :, pl.ds(0, WIN)])
        fc = process_window(win_v.at[0], cx, WIN, cnt, stage, fc)
        cy = pl.multiple_of(HBLK, 512)
        pltpu.sync_copy(emb_hbm.at[:, pl.ds(cy, 512)],
                        win_v.at[0, :, pl.ds(0, 512)])
        fc = process_window(win_v.at[0], cy, 512, cnt, stage, fc)
        fc = process_window(tail_ref, TAIL0, TAILW, cnt, stage, fc)
        flush(stage)

    sweep_table(uid_hbm, uemb_hbm, stage_u, tailu_v)
    sweep_table(mid_hbm, memb_hbm, stage_m, tailm_v)


@functools.partial(
    pl.kernel,
    mesh=_mesh,
    out_type=jax.ShapeDtypeStruct((B,), jnp.float32),
    compiler_params=pltpu.CompilerParams(use_tc_tiling_on_sc=False),
    scratch_types=[
        pltpu.VMEM((BPW,), jnp.int32),
        pltpu.VMEM((BPW,), jnp.int32),
        pltpu.VMEM((BPW,), jnp.float32),
        pltpu.VMEM((BPW,), jnp.float32),
        pltpu.SemaphoreType.DMA,
    ],
)
def _bias_kernel(uid_hbm, mid_hbm, ubias_hbm, mbias_hbm, out_hbm,
                 uid_v, mid_v, ub_v, mb_v, sem):
    wid = lax.axis_index("s") * NC + lax.axis_index("c")
    base = wid * BPW
    pltpu.sync_copy(uid_hbm.at[pl.ds(base, BPW)], uid_v)
    pltpu.sync_copy(mid_hbm.at[pl.ds(base, BPW)], mid_v)
    c1 = pltpu.async_copy(ubias_hbm.at[uid_v], ub_v, sem)
    c2 = pltpu.async_copy(mbias_hbm.at[mid_v], mb_v, sem)
    c1.wait()
    c2.wait()

    def body(g, _):
        sl = pl.ds(g * L, L)
        ub_v[sl] = ub_v[sl] + mb_v[sl]
        return 0

    lax.fori_loop(0, BPW // L, body, 0)
    pltpu.sync_copy(ub_v, out_hbm.at[pl.ds(base, BPW)])


_RCH = 128


@functools.partial(
    pl.kernel,
    mesh=_mesh,
    out_type=jax.ShapeDtypeStruct((B,), jnp.float32),
    compiler_params=pltpu.CompilerParams(use_tc_tiling_on_sc=True),
    scratch_types=[
        pltpu.VMEM((_RCH, 128), jnp.float32),
        pltpu.VMEM((_RCH, 128), jnp.float32),
        pltpu.VMEM((BPW,), jnp.float32),
        pltpu.VMEM((BPW,), jnp.float32),
    ],
)
def _dot_kernel(stage_u, stage_m, bias_hbm, out_hbm, su_v, sm_v, bias_v,
                out_v):
    wid = lax.axis_index("s") * NC + lax.axis_index("c")
    base = wid * BPW
    iot = _iota()
    pltpu.sync_copy(bias_hbm.at[pl.ds(base, BPW)], bias_v)
    lane_masks = [iot == j for j in range(L)]

    def chunk_body(ci, _):
        pltpu.sync_copy(stage_u.at[pl.ds(base + ci * _RCH, _RCH)], su_v)
        pltpu.sync_copy(stage_m.at[pl.ds(base + ci * _RCH, _RCH)], sm_v)

        def grp_body(g, _):
            acc = jnp.zeros((L,), jnp.float32)
            for j in range(L):
                r = g * L + j
                p = (su_v[r, pl.ds(0, L)] * sm_v[r, pl.ds(0, L)]
                     + su_v[r, pl.ds(L, L)] * sm_v[r, pl.ds(L, L)])
                for sh in (8, 4, 2, 1):
                    p = p + _lane_shuffle(p, iot ^ sh)
                acc = jnp.where(lane_masks[j], p, acc)
            sl = pl.ds(ci * _RCH + g * L, L)
            out_v[sl] = acc + bias_v[sl]
            return 0

        lax.fori_loop(0, _RCH // L, grp_body, 0)
        return 0

    lax.fori_loop(0, BPW // _RCH, chunk_body, 0)
    pltpu.sync_copy(out_v, out_hbm.at[pl.ds(base, BPW)])


def kernel(user_ids, movie_ids, user_embedding, movie_embedding,
           user_bias, movie_bias):
    ut = user_embedding.T
    mt = movie_embedding.T
    pad = ((0, 0), (0, 128 - TAILW))
    tail_u = jnp.pad(user_embedding[TAIL0:, :].T, pad)
    tail_m = jnp.pad(movie_embedding[TAIL0:, :].T, pad)
    stage_u, stage_m = _sweep_kernel(user_ids, movie_ids, ut, mt,
                                     tail_u, tail_m)
    bias_sum = _bias_kernel(user_ids, movie_ids, user_bias.reshape(-1),
                            movie_bias.reshape(-1))
    return _dot_kernel(stage_u, stage_m, bias_sum)


# speedup vs baseline: 1.0490x; 1.0186x over previous
import functools

import jax
import jax.numpy as jnp
from jax import lax
from jax.experimental import pallas as pl
from jax.experimental.pallas import tpu as pltpu
from jax.experimental.pallas import tpu_sc as plsc

L = 16
NC = 2
NS = 16
NW = NC * NS
B = 16384
D = 32
V = 1000000
BPW = B // NW

WIN = 1024
NFULL = V // WIN
WPW = NFULL // NW
NEXTRA = NFULL - WPW * NW
XBASE = WPW * WIN * NW
HBLK = XBASE + NEXTRA * WIN
TAIL0 = 999936
TAILW = V - TAIL0
STAGE_ROWS = B + 2 * NW

CHUNK = 2048
SEG = 1024

_mesh = plsc.VectorSubcoreMesh(core_axis_name="c", subcore_axis_name="s")

_GATHER_DNUMS = lax.GatherDimensionNumbers(
    offset_dims=(), collapsed_slice_dims=(0,), start_index_map=(0,))


def _lane_shuffle(x, perm):
    return lax.gather(x, perm[:, None], _GATHER_DNUMS, slice_sizes=(1,),
                      mode=lax.GatherScatterMode.PROMISE_IN_BOUNDS)


def _iota():
    return lax.iota(jnp.int32, L)


@functools.partial(
    pl.kernel,
    mesh=_mesh,
    out_type=(jax.ShapeDtypeStruct((STAGE_ROWS, 128), jnp.float32),
              jax.ShapeDtypeStruct((STAGE_ROWS, 128), jnp.float32)),
    compiler_params=pltpu.CompilerParams(use_tc_tiling_on_sc=True,
                                         needs_layout_passes=False),
    scratch_types=[
        pltpu.VMEM((2, CHUNK), jnp.int32),
        pltpu.VMEM((B + L,), jnp.int32),
        pltpu.VMEM((B + L,), jnp.int32),
        pltpu.VMEM((2, D, WIN + 1), jnp.float32),
        pltpu.VMEM((D, 128), jnp.float32),
        pltpu.VMEM((D, 128), jnp.float32),
        pltpu.VMEM((SEG + L,), jnp.int32),
        pltpu.VMEM((SEG + L,), jnp.int32),
        pltpu.VMEM((64, 128), jnp.float32),
        pltpu.VMEM((64,), jnp.int32),
        pltpu.SemaphoreType.DMA,
        pltpu.SemaphoreType.DMA,
        pltpu.SemaphoreType.DMA,
    ],
)
def _sweep_kernel(uid_hbm, mid_hbm, uemb_hbm, memb_hbm, tailu_hbm, tailm_hbm,
                  stage_u, stage_m, chunk_v, list_id, list_pos, win_v,
                  tailu_v, tailm_v, wl_id, wl_pos, rows_v, pos_v, wsem, ssem,
                  csem):
    wid = lax.axis_index("s") * NC + lax.axis_index("c")
    iot = _iota()
    lo = WPW * WIN * wid
    hi = lo + WPW * WIN
    xlo = jnp.where(wid < NEXTRA, XBASE + wid * WIN,
                    jnp.where(wid == NEXTRA, HBLK,
                              jnp.where(wid == NW - 1, TAIL0, 0)))
    xhi = jnp.where(wid < NEXTRA, XBASE + wid * WIN + WIN,
                    jnp.where(wid == NEXTRA, TAIL0,
                              jnp.where(wid == NW - 1, V, 0)))
    trash = B + wid

    pltpu.sync_copy(tailu_hbm, tailu_v)
    pltpu.sync_copy(tailm_hbm, tailm_v)

    def build_list(ids_hbm):
        for b in range(2):
            pltpu.async_copy(ids_hbm.at[pl.ds(b * CHUNK, CHUNK)],
                             chunk_v.at[b], csem)

        def chunk_pair(cp, off):
            for b in range(2):
                ci = cp * 2 + b
                pltpu.make_async_copy(ids_hbm.at[pl.ds(ci * CHUNK, CHUNK)],
                                      chunk_v.at[b], csem).wait()

                def vreg_body(i, off):
                    v = chunk_v[b, pl.ds(i * L, L)]
                    posv = ci * CHUNK + i * L + iot
                    m = ((v >= lo) & (v < hi)) | ((v >= xlo) & (v < xhi))
                    plsc.store_compressed(list_id.at[pl.ds(off, L)], v, mask=m)
                    plsc.store_compressed(list_pos.at[pl.ds(off, L)], posv,
                                          mask=m)
                    return off + plsc.all_reduce_population_count(m)[0]

                off = lax.fori_loop(0, CHUNK // L, vreg_body, off)
                ci2 = ci + 2

                @pl.when(ci2 < B // CHUNK)
                def _():
                    pltpu.async_copy(ids_hbm.at[pl.ds(ci2 * CHUNK, CHUNK)],
                                     chunk_v.at[b], csem)

            return off

        cnt = lax.fori_loop(0, B // CHUNK // 2, chunk_pair, 0)
        plsc.store_scatter(list_id, [cnt + iot], jnp.full((L,), -1, jnp.int32),
                           mask=None)
        return cnt

    def flush(stage):
        pltpu.async_copy(rows_v, stage.at[pos_v], ssem).wait()

    def process_window(src_v, c0, width, cnt, stage, fc):
        nseg = (cnt + (SEG - 1)) // SEG

        def seg_body(s, fc):
            sbase = s * SEG

            def sv(i, woff):
                gi = sbase + i * L
                v = list_id[pl.ds(gi, L)]
                p = list_pos[pl.ds(gi, L)]
                m = (v >= c0) & (v < c0 + width)
                plsc.store_compressed(wl_id.at[pl.ds(woff, L)], v - c0, mask=m)
                plsc.store_compressed(wl_pos.at[pl.ds(woff, L)], p, mask=m)
                return woff + plsc.all_reduce_population_count(m)[0]

            nv = (jnp.minimum(cnt - sbase, SEG) + (L - 1)) // L
            wcnt = lax.fori_loop(0, nv, sv, 0)
            plsc.store_scatter(wl_id, [wcnt + iot], jnp.zeros((L,), jnp.int32),
                               mask=None)
            plsc.store_scatter(wl_pos, [wcnt + iot],
                               jnp.full((L,), trash, jnp.int32), mask=None)

            def grp_body(g, fc):
                wc16 = wl_id[pl.ds(g * L, L)]
                pos16 = wl_pos[pl.ds(g * L, L)]
                pos_v[pl.ds(fc * L, L)] = pos16
                for j in range(L):
                    wcj = _lane_shuffle(wc16, jnp.full((L,), j, jnp.int32))
                    g0 = plsc.load_gather(src_v, [iot, wcj])
                    g1 = plsc.load_gather(src_v, [iot + L, wcj])
                    r = fc * L + j
                    rows_v[r, pl.ds(0, L)] = g0
                    rows_v[r, pl.ds(L, L)] = g1

                @pl.when(fc == 3)
                def _():
                    flush(stage)

                return (fc + 1) & 3

            ngrp = (wcnt + (L - 1)) // L
            return lax.fori_loop(0, ngrp, grp_body, fc)

        return lax.fori_loop(0, nseg, seg_body, fc)

    def sweep_table(ids_hbm, emb_hbm, stage, tail_ref):
        for q in range(4):
            pos_v[pl.ds(q * L, L)] = jnp.full((L,), trash, jnp.int32)

        for b in range(2):
            cb = pl.multiple_of(lo + b * WIN, WIN)
            pltpu.async_copy(emb_hbm.at[:, pl.ds(cb, WIN)],
                             win_v.at[b, :, pl.ds(0, WIN)], wsem)

        cnt = build_list(ids_hbm)

        def pair_body(p, fc):
            for b in range(2):
                k = 2 * p + b
                cw = pl.multiple_of(lo + k * WIN, WIN)
                pltpu.make_async_copy(emb_hbm.at[:, pl.ds(cw, WIN)],
                                      win_v.at[b, :, pl.ds(0, WIN)],
                                      wsem).wait()
                fc = process_window(win_v.at[b], cw, WIN, cnt, stage, fc)

                k2 = k + 2

                @pl.when(k2 < WPW)
                def _():
                    c2 = pl.multiple_of(lo + k2 * WIN, WIN)
                    pltpu.async_copy(emb_hbm.at[:, pl.ds(c2, WIN)],
                                     win_v.at[b, :, pl.ds(0, WIN)], wsem)

            return fc

        fc = lax.fori_loop(0, WPW // 2, pair_body, jnp.int32(0))

        cx = pl.multiple_of(
            jnp.where(wid < NEXTRA, XBASE + wid * WIN, XBASE), WIN)
        pltpu.sync_copy(emb_hbm.at[:, pl.ds(cx, WIN)],
                        win_v.at[0, :, pl.ds(0, WIN)])
        fc = process_window(win_v.at[0], cx, WIN, cnt, stage, fc)
        cy = pl.multiple_of(HBLK, 512)
        pltpu.sync_copy(emb_hbm.at[:, pl.ds(cy, 512)],
                        win_v.at[0, :, pl.ds(0, 512)])
        fc = process_window(win_v.at[0], cy, 512, cnt, stage, fc)
        fc = process_window(tail_ref, TAIL0, TAILW, cnt, stage, fc)
        flush(stage)

    sweep_table(uid_hbm, uemb_hbm, stage_u, tailu_v)
    sweep_table(mid_hbm, memb_hbm, stage_m, tailm_v)


@functools.partial(
    pl.kernel,
    mesh=_mesh,
    out_type=jax.ShapeDtypeStruct((B,), jnp.float32),
    compiler_params=pltpu.CompilerParams(use_tc_tiling_on_sc=False),
    scratch_types=[
        pltpu.VMEM((BPW,), jnp.int32),
        pltpu.VMEM((BPW,), jnp.int32),
        pltpu.VMEM((BPW,), jnp.float32),
        pltpu.VMEM((BPW,), jnp.float32),
        pltpu.SemaphoreType.DMA,
    ],
)
def _bias_kernel(uid_hbm, mid_hbm, ubias_hbm, mbias_hbm, out_hbm,
                 uid_v, mid_v, ub_v, mb_v, sem):
    wid = lax.axis_index("s") * NC + lax.axis_index("c")
    base = wid * BPW
    pltpu.sync_copy(uid_hbm.at[pl.ds(base, BPW)], uid_v)
    pltpu.sync_copy(mid_hbm.at[pl.ds(base, BPW)], mid_v)
    c1 = pltpu.async_copy(ubias_hbm.at[uid_v], ub_v, sem)
    c2 = pltpu.async_copy(mbias_hbm.at[mid_v], mb_v, sem)
    c1.wait()
    c2.wait()

    def body(g, _):
        sl = pl.ds(g * L, L)
        ub_v[sl] = ub_v[sl] + mb_v[sl]
        return 0

    lax.fori_loop(0, BPW // L, body, 0)
    pltpu.sync_copy(ub_v, out_hbm.at[pl.ds(base, BPW)])


_RCH = 128


@functools.partial(
    pl.kernel,
    mesh=_mesh,
    out_type=jax.ShapeDtypeStruct((B,), jnp.float32),
    compiler_params=pltpu.CompilerParams(use_tc_tiling_on_sc=True),
    scratch_types=[
        pltpu.VMEM((2, _RCH, 128), jnp.float32),
        pltpu.VMEM((2, _RCH, 128), jnp.float32),
        pltpu.VMEM((BPW,), jnp.float32),
        pltpu.VMEM((BPW,), jnp.float32),
        pltpu.SemaphoreType.DMA,
    ],
)
def _dot_kernel(stage_u, stage_m, bias_hbm, out_hbm, su2_v, sm2_v, bias_v,
                out_v, dsem):
    wid = lax.axis_index("s") * NC + lax.axis_index("c")
    base = wid * BPW
    iot = _iota()
    for b in range(2):
        sl0 = pl.ds(base + b * _RCH, _RCH)
        pltpu.async_copy(stage_u.at[sl0], su2_v.at[b], dsem)
        pltpu.async_copy(stage_m.at[sl0], sm2_v.at[b], dsem)
    pltpu.sync_copy(bias_hbm.at[pl.ds(base, BPW)], bias_v)
    lane_masks = [iot == j for j in range(L)]

    def pair_body(cp, _):
        for b in range(2):
            ci = cp * 2 + b
            su_v = su2_v.at[b]
            sm_v = sm2_v.at[b]
            slc = pl.ds(base + ci * _RCH, _RCH)
            pltpu.make_async_copy(stage_u.at[slc], su2_v.at[b], dsem).wait()
            pltpu.make_async_copy(stage_m.at[slc], sm2_v.at[b], dsem).wait()

            def grp_body(g, _):
                acc = jnp.zeros((L,), jnp.float32)
                for j in range(L):
                    r = g * L + j
                    p = (su_v[r, pl.ds(0, L)] * sm_v[r, pl.ds(0, L)]
                         + su_v[r, pl.ds(L, L)] * sm_v[r, pl.ds(L, L)])
                    for sh in (8, 4, 2, 1):
                        p = p + _lane_shuffle(p, iot ^ sh)
                    acc = jnp.where(lane_masks[j], p, acc)
                sl = pl.ds(ci * _RCH + g * L, L)
                out_v[sl] = acc + bias_v[sl]
                return 0

            lax.fori_loop(0, _RCH // L, grp_body, 0)
            ci2 = ci + 2

            @pl.when(ci2 < BPW // _RCH)
            def _():
                sl2 = pl.ds(base + ci2 * _RCH, _RCH)
                pltpu.async_copy(stage_u.at[sl2], su2_v.at[b], dsem)
                pltpu.async_copy(stage_m.at[sl2], sm2_v.at[b], dsem)

        return 0

    lax.fori_loop(0, BPW // _RCH // 2, pair_body, 0)
    pltpu.sync_copy(out_v, out_hbm.at[pl.ds(base, BPW)])


def kernel(user_ids, movie_ids, user_embedding, movie_embedding,
           user_bias, movie_bias):
    ut = user_embedding.T
    mt = movie_embedding.T
    pad = ((0, 0), (0, 128 - TAILW))
    tail_u = jnp.pad(user_embedding[TAIL0:, :].T, pad)
    tail_m = jnp.pad(movie_embedding[TAIL0:, :].T, pad)
    stage_u, stage_m = _sweep_kernel(user_ids, movie_ids, ut, mt,
                                     tail_u, tail_m)
    bias_sum = _bias_kernel(user_ids, movie_ids, user_bias.reshape(-1),
                            movie_bias.reshape(-1))
    return _dot_kernel(stage_u, stage_m, bias_sum)
